# baseline (speedup 1.0000x reference)
import functools

import jax
import jax.numpy as jnp
from jax.experimental import pallas as pl
from jax.experimental.pallas import tpu as pltpu
from jax.experimental.pallas import tpu_sc as plsc

G, V = 8, 512
GV = G * V
D = 128
TT = 512
NC, NS = 2, 16
NW = NC * NS
CH = 128


def _gumbel_z(l, temp):
    gum = -jnp.log(-jnp.log(l + 1e-05) + 1e-05)
    return (l + gum) / temp


def _logits_kernel(temp_ref, x_ref, w_ref, b_ref, logits_ref, idx_ref):
    x = x_ref[0]
    temp = temp_ref[0]
    logits = jax.lax.dot_general(
        w_ref[...], x, (((1,), (1,)), ((), ())),
        preferred_element_type=jnp.float32)
    logits = logits + b_ref[...]
    logits_ref[0] = logits
    idxs = []
    for g in range(G):
        lg = logits[g * V:(g + 1) * V, :]
        j1 = jnp.argmax(lg, axis=0)
        m1 = jnp.max(lg, axis=0)
        masked = jnp.where(lg == m1[None, :], -jnp.inf, lg)
        j2 = jnp.argmax(masked, axis=0)
        m2 = jnp.max(masked, axis=0)
        collapse = jnp.exp(_gumbel_z(m2, temp) - _gumbel_z(m1, temp)) >= 1.0
        idx = jnp.where(collapse, jnp.minimum(j1, j2), j1)
        idxs.append(idx + g * V)
    idx_ref[0] = jnp.stack(idxs, axis=0)


def _gather_kernel(table_hbm, idx_hbm, out_hbm, idx_v, rows_v, sem):
    w = jax.lax.axis_index("s") * NC + jax.lax.axis_index("c")
    b = w // G
    g = w % G
    t_total = idx_hbm.shape[2]
    for c in range(t_total // CH):
        pltpu.sync_copy(idx_hbm.at[b, g, pl.ds(c * CH, CH)], idx_v)
        pltpu.async_copy(table_hbm.at[idx_v], rows_v, sem).wait()
        pltpu.sync_copy(rows_v, out_hbm.at[b, pl.ds(c * CH, CH), g])


def kernel(inputs, W, b, codebooks, temp):
    bsize, timesteps, channels = inputs.shape
    b2 = b.reshape(GV, 1)
    temp1 = jnp.asarray(temp, jnp.float32).reshape(1)
    table = jnp.transpose(codebooks.reshape(G, D, V), (0, 2, 1)).reshape(GV, D)
    logits_flat, idx = pl.pallas_call(
        _logits_kernel,
        grid=(bsize, timesteps // TT),
        in_specs=[
            pl.BlockSpec(memory_space=pltpu.SMEM),
            pl.BlockSpec((1, TT, channels), lambda i, j: (i, j, 0)),
            pl.BlockSpec((GV, channels), lambda i, j: (0, 0)),
            pl.BlockSpec((GV, 1), lambda i, j: (0, 0)),
        ],
        out_specs=[
            pl.BlockSpec((1, GV, TT), lambda i, j: (i, 0, j)),
            pl.BlockSpec((1, G, TT), lambda i, j: (i, 0, j)),
        ],
        out_shape=[
            jax.ShapeDtypeStruct((bsize, GV, timesteps), jnp.float32),
            jax.ShapeDtypeStruct((bsize, G, timesteps), jnp.int32),
        ],
        compiler_params=pltpu.CompilerParams(
            vmem_limit_bytes=64 * 1024 * 1024),
    )(temp1, inputs, W, b2)
    logits = logits_flat.reshape(bsize, G, V, timesteps)

    mesh = plsc.VectorSubcoreMesh(core_axis_name="c", subcore_axis_name="s")
    gather = functools.partial(
        pl.kernel,
        mesh=mesh,
        out_type=jax.ShapeDtypeStruct((bsize, timesteps, G, D), jnp.float32),
        scratch_types=[
            pltpu.VMEM((CH,), jnp.int32),
            pltpu.VMEM((CH, D), jnp.float32),
            pltpu.SemaphoreType.DMA,
        ],
    )(_gather_kernel)
    out4 = gather(table, idx)
    out = out4.reshape(bsize, timesteps, channels)
    return out, logits


# speedup vs baseline: 1.0140x; 1.0140x over previous
import functools

import jax
import jax.numpy as jnp
from jax.experimental import pallas as pl
from jax.experimental.pallas import tpu as pltpu
from jax.experimental.pallas import tpu_sc as plsc

G, V = 8, 512
GV = G * V
D = 128
TT = 512
NC, NS = 2, 16
NW = NC * NS
CH = 128


def _gumbel_z(l, temp):
    gum = -jnp.log(-jnp.log(l + 1e-05) + 1e-05)
    return (l + gum) / temp


def _logits_kernel(temp_ref, x_ref, w_ref, b_ref, logits_ref, idx_ref):
    x = x_ref[0]
    temp = temp_ref[0]
    logits = jax.lax.dot_general(
        w_ref[...], x, (((1,), (1,)), ((), ())),
        preferred_element_type=jnp.float32)
    logits = logits + b_ref[...]
    logits_ref[0] = logits
    idxs = []
    for g in range(G):
        lg = logits[g * V:(g + 1) * V, :]
        j1 = jnp.argmax(lg, axis=0)
        m1 = jnp.max(lg, axis=0)
        masked = jnp.where(lg == m1[None, :], -jnp.inf, lg)
        j2 = jnp.argmax(masked, axis=0)
        m2 = jnp.max(masked, axis=0)
        collapse = jnp.exp(_gumbel_z(m2, temp) - _gumbel_z(m1, temp)) >= 1.0
        idx = jnp.where(collapse, jnp.minimum(j1, j2), j1)
        idxs.append(idx + g * V)
    idx_ref[0] = jnp.stack(idxs, axis=0)


def _gather_kernel(table_hbm, idx_hbm, out_hbm, idx_v, rows_v, sem):
    w = jax.lax.axis_index("s") * NC + jax.lax.axis_index("c")
    b = w // G
    g = w % G
    nch = idx_hbm.shape[2]
    pltpu.sync_copy(idx_hbm.at[b, g], idx_v)
    grp = 4
    for m in range(nch // grp):
        cps = [
            pltpu.async_copy(
                table_hbm.at[idx_v.at[m * grp + j]],
                rows_v.at[pl.ds(j * CH, CH)], sem)
            for j in range(grp)
        ]
        for cp in cps:
            cp.wait()
        pltpu.sync_copy(
            rows_v, out_hbm.at[b, pl.ds(m * grp * CH, grp * CH), g])


def kernel(inputs, W, b, codebooks, temp):
    bsize, timesteps, channels = inputs.shape
    b2 = b.reshape(GV, 1)
    temp1 = jnp.asarray(temp, jnp.float32).reshape(1)
    table = jnp.transpose(codebooks.reshape(G, D, V), (0, 2, 1)).reshape(GV, D)
    logits_flat, idx = pl.pallas_call(
        _logits_kernel,
        grid=(bsize, timesteps // TT),
        in_specs=[
            pl.BlockSpec(memory_space=pltpu.SMEM),
            pl.BlockSpec((1, TT, channels), lambda i, j: (i, j, 0)),
            pl.BlockSpec((GV, channels), lambda i, j: (0, 0)),
            pl.BlockSpec((GV, 1), lambda i, j: (0, 0)),
        ],
        out_specs=[
            pl.BlockSpec((1, GV, TT), lambda i, j: (i, 0, j)),
            pl.BlockSpec((1, G, TT), lambda i, j: (i, 0, j)),
        ],
        out_shape=[
            jax.ShapeDtypeStruct((bsize, GV, timesteps), jnp.float32),
            jax.ShapeDtypeStruct((bsize, G, timesteps), jnp.int32),
        ],
        compiler_params=pltpu.CompilerParams(
            vmem_limit_bytes=64 * 1024 * 1024),
    )(temp1, inputs, W, b2)
    logits = logits_flat.reshape(bsize, G, V, timesteps)

    mesh = plsc.VectorSubcoreMesh(core_axis_name="c", subcore_axis_name="s")
    gather = functools.partial(
        pl.kernel,
        mesh=mesh,
        out_type=jax.ShapeDtypeStruct(
            (bsize, timesteps, G, D), jnp.float32),
        scratch_types=[
            pltpu.VMEM((timesteps // CH, CH), jnp.int32),
            pltpu.VMEM((4 * CH, D), jnp.float32),
            pltpu.SemaphoreType.DMA,
        ],
    )(_gather_kernel)
    out4 = gather(table, idx.reshape(bsize, G, timesteps // CH, CH))
    out = out4.reshape(bsize, timesteps, channels)
    return out, logits


# speedup vs baseline: 2.2788x; 2.2474x over previous
import functools

import jax
import jax.numpy as jnp
from jax.experimental import pallas as pl
from jax.experimental.pallas import tpu as pltpu
from jax.experimental.pallas import tpu_sc as plsc

G, V = 8, 512
GV = G * V
D = 128
TT = 512
NC, NS = 2, 16
NW = NC * NS
CH = 128


def _gumbel_z(l, temp):
    gum = -jnp.log(-jnp.log(l + 1e-05) + 1e-05)
    return (l + gum) / temp


def _logits_kernel(temp_ref, x_ref, w_ref, b_ref, logits_ref, idx_ref):
    x = x_ref[0]
    temp = temp_ref[0]
    logits = jax.lax.dot_general(
        w_ref[...], x, (((1,), (1,)), ((), ())),
        preferred_element_type=jnp.float32)
    logits = logits + b_ref[...]
    logits_ref[0] = logits
    idxs = []
    for g in range(G):
        lg = logits[g * V:(g + 1) * V, :]
        j1 = jnp.argmax(lg, axis=0)
        m1 = jnp.max(lg, axis=0)
        masked = jnp.where(lg == m1[None, :], -jnp.inf, lg)
        j2 = jnp.argmax(masked, axis=0)
        m2 = jnp.max(masked, axis=0)
        collapse = jnp.exp(_gumbel_z(m2, temp) - _gumbel_z(m1, temp)) >= 1.0
        idx = jnp.where(collapse, jnp.minimum(j1, j2), j1)
        idxs.append(idx + g * V)
    idx_ref[0] = jnp.stack(idxs, axis=0)


def _gather_kernel(table_hbm, idx_hbm, out_hbm, idx_v, rows_v, shared, sem):
    w = jax.lax.axis_index("s") * NC + jax.lax.axis_index("c")
    b = w // G
    g = w % G
    sid = jax.lax.axis_index("s")
    stripe = GV // NS
    pltpu.sync_copy(table_hbm.at[pl.ds(sid * stripe, stripe)],
                    shared.at[pl.ds(sid * stripe, stripe)])
    plsc.subcore_barrier()
    nch = idx_hbm.shape[2]
    pltpu.sync_copy(idx_hbm.at[b, g], idx_v)
    grp = 4
    for m in range(nch // grp):
        cps = [
            pltpu.async_copy(
                shared.at[idx_v.at[m * grp + j]],
                rows_v.at[pl.ds(j * CH, CH)], sem)
            for j in range(grp)
        ]
        for cp in cps:
            cp.wait()
        pltpu.sync_copy(
            rows_v, out_hbm.at[b, pl.ds(m * grp * CH, grp * CH), g])


def kernel(inputs, W, b, codebooks, temp):
    bsize, timesteps, channels = inputs.shape
    b2 = b.reshape(GV, 1)
    temp1 = jnp.asarray(temp, jnp.float32).reshape(1)
    table = jnp.transpose(codebooks.reshape(G, D, V), (0, 2, 1)).reshape(GV, D)
    logits_flat, idx = pl.pallas_call(
        _logits_kernel,
        grid=(bsize, timesteps // TT),
        in_specs=[
            pl.BlockSpec(memory_space=pltpu.SMEM),
            pl.BlockSpec((1, TT, channels), lambda i, j: (i, j, 0)),
            pl.BlockSpec((GV, channels), lambda i, j: (0, 0)),
            pl.BlockSpec((GV, 1), lambda i, j: (0, 0)),
        ],
        out_specs=[
            pl.BlockSpec((1, GV, TT), lambda i, j: (i, 0, j)),
            pl.BlockSpec((1, G, TT), lambda i, j: (i, 0, j)),
        ],
        out_shape=[
            jax.ShapeDtypeStruct((bsize, GV, timesteps), jnp.float32),
            jax.ShapeDtypeStruct((bsize, G, timesteps), jnp.int32),
        ],
        compiler_params=pltpu.CompilerParams(
            vmem_limit_bytes=64 * 1024 * 1024),
    )(temp1, inputs, W, b2)
    logits = logits_flat.reshape(bsize, G, V, timesteps)

    mesh = plsc.VectorSubcoreMesh(core_axis_name="c", subcore_axis_name="s")
    gather = functools.partial(
        pl.kernel,
        mesh=mesh,
        out_type=jax.ShapeDtypeStruct(
            (bsize, timesteps, G, D), jnp.float32),
        scratch_types=[
            pltpu.VMEM((timesteps // CH, CH), jnp.int32),
            pltpu.VMEM((4 * CH, D), jnp.float32),
            pltpu.VMEM_SHARED((GV, D), jnp.float32),
            pltpu.SemaphoreType.DMA,
        ],
    )(_gather_kernel)
    out4 = gather(table, idx.reshape(bsize, G, timesteps // CH, CH))
    out = out4.reshape(bsize, timesteps, channels)
    return out, logits


# speedup vs baseline: 2.3207x; 1.0184x over previous
import functools

import jax
import jax.numpy as jnp
from jax.experimental import pallas as pl
from jax.experimental.pallas import tpu as pltpu
from jax.experimental.pallas import tpu_sc as plsc

G, V = 8, 512
GV = G * V
D = 128
TT = 512
NC, NS = 2, 16
NW = NC * NS
CH = 128


def _gumbel_z(l, temp):
    gum = -jnp.log(-jnp.log(l + 1e-05) + 1e-05)
    return (l + gum) / temp


def _logits_kernel(temp_ref, x_ref, w_ref, b_ref, logits_ref, idx_ref):
    x = x_ref[0]
    temp = temp_ref[0]
    logits = jax.lax.dot_general(
        w_ref[...], x, (((1,), (1,)), ((), ())),
        preferred_element_type=jnp.float32)
    logits = logits + b_ref[...]
    logits_ref[0] = logits
    idxs = []
    for g in range(G):
        lg = logits[g * V:(g + 1) * V, :]
        j1 = jnp.argmax(lg, axis=0)
        m1 = jnp.max(lg, axis=0)
        masked = jnp.where(lg == m1[None, :], -jnp.inf, lg)
        j2 = jnp.argmax(masked, axis=0)
        m2 = jnp.max(masked, axis=0)
        collapse = jnp.exp(_gumbel_z(m2, temp) - _gumbel_z(m1, temp)) >= 1.0
        idx = jnp.where(collapse, jnp.minimum(j1, j2), j1)
        idxs.append(idx + g * V)
    idx_ref[0] = jnp.stack(idxs, axis=0)


def _gather_kernel(table_hbm, idx_hbm, out_hbm, idx_v, ra, rb, shared,
                   gsem, sema, semb):
    w = jax.lax.axis_index("s") * NC + jax.lax.axis_index("c")
    b = w // G
    g = w % G
    sid = jax.lax.axis_index("s")
    stripe = GV // NS
    pltpu.sync_copy(table_hbm.at[pl.ds(sid * stripe, stripe)],
                    shared.at[pl.ds(sid * stripe, stripe)])
    plsc.subcore_barrier()
    nch = idx_hbm.shape[2]
    pltpu.sync_copy(idx_hbm.at[b, g], idx_v)
    grp = 2
    bufs = (ra, rb)
    ssems = (sema, semb)
    store_cps = [None, None]
    for m in range(nch // grp):
        buf = bufs[m % 2]
        if store_cps[m % 2] is not None:
            store_cps[m % 2].wait()
        cps = [
            pltpu.async_copy(
                shared.at[idx_v.at[m * grp + j]],
                buf.at[pl.ds(j * CH, CH)], gsem)
            for j in range(grp)
        ]
        for cp in cps:
            cp.wait()
        store_cps[m % 2] = pltpu.async_copy(
            buf, out_hbm.at[b, pl.ds(m * grp * CH, grp * CH), g],
            ssems[m % 2])
    for cp in store_cps:
        cp.wait()


def kernel(inputs, W, b, codebooks, temp):
    bsize, timesteps, channels = inputs.shape
    b2 = b.reshape(GV, 1)
    temp1 = jnp.asarray(temp, jnp.float32).reshape(1)
    table = jnp.transpose(codebooks.reshape(G, D, V), (0, 2, 1)).reshape(GV, D)
    logits_flat, idx = pl.pallas_call(
        _logits_kernel,
        grid=(bsize, timesteps // TT),
        in_specs=[
            pl.BlockSpec(memory_space=pltpu.SMEM),
            pl.BlockSpec((1, TT, channels), lambda i, j: (i, j, 0)),
            pl.BlockSpec((GV, channels), lambda i, j: (0, 0)),
            pl.BlockSpec((GV, 1), lambda i, j: (0, 0)),
        ],
        out_specs=[
            pl.BlockSpec((1, GV, TT), lambda i, j: (i, 0, j)),
            pl.BlockSpec((1, G, TT), lambda i, j: (i, 0, j)),
        ],
        out_shape=[
            jax.ShapeDtypeStruct((bsize, GV, timesteps), jnp.float32),
            jax.ShapeDtypeStruct((bsize, G, timesteps), jnp.int32),
        ],
        compiler_params=pltpu.CompilerParams(
            vmem_limit_bytes=64 * 1024 * 1024),
    )(temp1, inputs, W, b2)
    logits = logits_flat.reshape(bsize, G, V, timesteps)

    mesh = plsc.VectorSubcoreMesh(core_axis_name="c", subcore_axis_name="s")
    gather = functools.partial(
        pl.kernel,
        mesh=mesh,
        out_type=jax.ShapeDtypeStruct(
            (bsize, timesteps, G, D), jnp.float32),
        scratch_types=[
            pltpu.VMEM((timesteps // CH, CH), jnp.int32),
            pltpu.VMEM((2 * CH, D), jnp.float32),
            pltpu.VMEM((2 * CH, D), jnp.float32),
            pltpu.VMEM_SHARED((GV, D), jnp.float32),
            pltpu.SemaphoreType.DMA,
            pltpu.SemaphoreType.DMA,
            pltpu.SemaphoreType.DMA,
        ],
    )(_gather_kernel)
    out4 = gather(table, idx.reshape(bsize, G, timesteps // CH, CH))
    out = out4.reshape(bsize, timesteps, channels)
    return out, logits
